# baseline (speedup 1.0000x reference)
import functools

import jax
import jax.numpy as jnp
from jax import lax
from jax.experimental import pallas as pl
from jax.experimental.pallas import tpu as pltpu
from jax.experimental.pallas import tpu_sc as plsc

N = 100000
E = 6400000
DT = 0.1
EPS = 1e-5

LANES = 128
GROUPS = E // LANES
N_PAD = 100352
ROWS_PAD = N_PAD // 128
G_CHUNK = 8
N_CHUNKS = 196
NBUF = 4
SLICE = N_PAD // 16
NPIECE = 8
QSLICE = SLICE // NPIECE
HSLICE = QSLICE // 2


def _pack_body(src_ref, dst_ref, th_ref, cf_ref, dl_ref, out_ref):
    w = jnp.tanh(th_ref[...]) * cf_ref[...] * dl_ref[...]
    out_ref[:, 0, :] = src_ref[...]
    out_ref[:, 1, :] = dst_ref[...]
    out_ref[:, 2, :] = lax.bitcast_convert_type(w, jnp.int32)


def _pack_edata(src2, dst2, theta2, conf2, delay2):
    blk = 2000
    spec = pl.BlockSpec((blk, LANES), lambda i: (i, 0))
    return pl.pallas_call(
        _pack_body,
        grid=(GROUPS // blk,),
        in_specs=[spec] * 5,
        out_specs=pl.BlockSpec((blk, 3, LANES), lambda i: (i, 0, 0)),
        out_shape=jax.ShapeDtypeStruct((GROUPS, 3, LANES), jnp.int32),
    )(src2, dst2, theta2, conf2, delay2)


def _update_body(state_ref, p0_ref, p1_ref, bias_ref, rls_ref, base_ref,
                 cap_ref, out_ref):
    s = state_ref[...]
    infl = p0_ref[...] + p1_ref[...]
    drive = jnp.tanh(infl + bias_ref[...])
    rate = base_ref[...] * jnp.exp(rls_ref[...])
    cap = cap_ref[...]
    dx = rate * drive * s * (1.0 - s / jnp.clip(cap, EPS))
    out_ref[...] = jnp.clip(s + DT * dx, 0.0, cap)


def _node_update(state2, p0, p1, bias2, rls2, base2, cap2):
    return pl.pallas_call(
        _update_body,
        out_shape=jax.ShapeDtypeStruct((ROWS_PAD, 128), jnp.float32),
    )(state2, p0, p1, bias2, rls2, base2, cap2)


def _make_edge_body(with_update):
    def body(*refs):
        if with_update:
            (state_hbm, parts_hbm, edata_hbm, bias_hbm, base_hbm, rls_hbm,
             cap_hbm, stateout_hbm, partsout_hbm,
             state_v, ring_v, msg_v, stage_v,
             u_s, u_p0, u_p1, u_bias, u_base, u_rls, u_cap, infl_s,
             sem_state, sem_in0, sem_in1, sem_in2, sem_in3,
             sem_sc0, sem_sc1, sem_sc2, sem_sc3) = refs
        else:
            (state_hbm, edata_hbm, partsout_hbm,
             state_v, ring_v, msg_v, stage_v, infl_s,
             sem_state, sem_in0, sem_in1, sem_in2, sem_in3,
             sem_sc0, sem_sc1, sem_sc2, sem_sc3) = refs
        sem_in = [sem_in0, sem_in1, sem_in2, sem_in3]
        sem_sc = [sem_sc0, sem_sc1, sem_sc2, sem_sc3]
        cid = lax.axis_index("c")
        sid = lax.axis_index("s")
        wid = sid * 2 + cid
        n_g = jnp.where(wid < 10, 1568, 1560)
        base_g = 8 * (195 * wid + jnp.minimum(wid, 10))

        def _row0(c):
            return base_g + jnp.minimum(G_CHUNK * c, n_g - G_CHUNK)

        def _fire_in(c, b):
            pltpu.async_copy(edata_hbm.at[pl.ds(_row0(c) * 3, 3 * G_CHUNK)],
                             ring_v.at[b], sem_in[b])

        def _wait_in(c, b):
            pltpu.make_async_copy(edata_hbm.at[pl.ds(_row0(c) * 3, 3 * G_CHUNK)],
                                  ring_v.at[b], sem_in[b]).wait()

        def _fire_sc(b):
            for j in range(G_CHUNK):
                pltpu.async_copy(msg_v.at[b, j],
                                 infl_s.at[ring_v.at[b, 3 * j + 1]],
                                 sem_sc[b], add=True)

        def _drain_sc(b):
            for j in range(G_CHUNK):
                pltpu.make_async_copy(msg_v.at[b, j],
                                      infl_s.at[ring_v.at[b, 3 * j + 1]],
                                      sem_sc[b]).wait()

        if not with_update:
            state_cp = pltpu.async_copy(state_hbm, state_v, sem_state)

        zeros16 = jnp.zeros((16,), jnp.float32)

        def _zero_body(i, carry):
            stage_v[pl.ds(i * 16, 16)] = zeros16
            return carry

        def _zero_infl():
            lax.fori_loop(0, QSLICE // 16, _zero_body, 0)
            for q in range(NPIECE):
                pltpu.sync_copy(
                    stage_v,
                    infl_s.at[pl.ds(sid * SLICE + q * QSLICE, QSLICE)])

        if with_update:
            state_s = infl_s
            for q in range(NPIECE):
                off = sid * SLICE + q * QSLICE
                sl_q = pl.ds(off, QSLICE)
                cps = [
                    pltpu.async_copy(state_hbm.at[sl_q], u_s, sem_state),
                    pltpu.async_copy(parts_hbm.at[sl_q], u_p0, sem_state),
                    pltpu.async_copy(parts_hbm.at[pl.ds(N_PAD + off, QSLICE)],
                                     u_p1, sem_state),
                    pltpu.async_copy(bias_hbm.at[sl_q], u_bias, sem_state),
                    pltpu.async_copy(base_hbm.at[sl_q], u_base, sem_state),
                    pltpu.async_copy(rls_hbm.at[sl_q], u_rls, sem_state),
                    pltpu.async_copy(cap_hbm.at[sl_q], u_cap, sem_state),
                ]
                for cp in cps:
                    cp.wait()
                def _upd_body(i, carry):
                    sl = pl.ds(i * 16, 16)
                    s = u_s[sl]
                    y = u_p0[sl] + u_p1[sl] + u_bias[sl]
                    t = jnp.exp(-2.0 * jnp.abs(y))
                    mag = (1.0 - t) / (1.0 + t)
                    drive = jnp.where(y >= 0.0, mag, -mag)
                    rate = u_base[sl] * jnp.exp(u_rls[sl])
                    cap = u_cap[sl]
                    dx = rate * drive * s * (1.0 - s / jnp.maximum(cap, EPS))
                    stage_v[sl] = jnp.clip(s + DT * dx, 0.0, cap)
                    return carry

                lax.fori_loop(0, QSLICE // 16, _upd_body, 0)
                pltpu.sync_copy(stage_v, state_s.at[sl_q])
                pltpu.sync_copy(
                    stage_v.at[pl.ds(cid * HSLICE, HSLICE)],
                    stateout_hbm.at[pl.ds(off + cid * HSLICE, HSLICE)])
            plsc.subcore_barrier()
            pltpu.sync_copy(state_s, state_v)
            plsc.subcore_barrier()
            _zero_infl()
            plsc.subcore_barrier()
        else:
            _zero_infl()
            state_cp.wait()
            plsc.subcore_barrier()

        _fire_in(0, 0)
        _fire_in(1, 1)

        def _chunk_body(p, carry):
            for b in range(NBUF):
                c = NBUF * p + b
                bg = jnp.minimum(G_CHUNK * c, n_g - G_CHUNK)
                _wait_in(c, b)
                for j in range(G_CHUNK):
                    valid = (bg + j >= G_CHUNK * c).astype(jnp.float32)
                    for k in range(LANES // 16):
                        sl = pl.ds(k * 16, 16)
                        idx = ring_v[b, 3 * j, sl]
                        wv = plsc.bitcast(ring_v[b, 3 * j + 2, sl], jnp.float32)
                        vals = plsc.load_gather(state_v, [idx])
                        msg_v[b, j, sl] = vals * wv * valid
                b2 = (b + 2) % NBUF

                @pl.when(c >= 2)
                def _():
                    _drain_sc(b2)

                _fire_sc(b)

                @pl.when(c + 2 <= N_CHUNKS - 1)
                def _():
                    _fire_in(c + 2, b2)
            return carry

        lax.fori_loop(0, N_CHUNKS // NBUF, _chunk_body, 0)
        _drain_sc(2)
        _drain_sc(3)
        plsc.subcore_barrier()

        for q in range(NPIECE):
            off = sid * SLICE + q * QSLICE
            pltpu.sync_copy(infl_s.at[pl.ds(off, QSLICE)], stage_v)
            pltpu.sync_copy(stage_v,
                            partsout_hbm.at[pl.ds(cid * N_PAD + off, QSLICE)])

    return body


_SC_MESH = plsc.VectorSubcoreMesh(core_axis_name="c", subcore_axis_name="s")
_SC_PARAMS = pltpu.CompilerParams(needs_layout_passes=False)
_COMMON_SCRATCH = [
    pltpu.VMEM((N_PAD,), jnp.float32),
    pltpu.VMEM((NBUF, 3 * G_CHUNK, LANES), jnp.int32),
    pltpu.VMEM((NBUF, G_CHUNK, LANES), jnp.float32),
    pltpu.VMEM((QSLICE,), jnp.float32),
]
_SEMS = [pltpu.SemaphoreType.DMA] * 9

_edge_call0 = functools.partial(
    pl.kernel,
    out_type=jax.ShapeDtypeStruct((2 * N_PAD,), jnp.float32),
    mesh=_SC_MESH,
    compiler_params=_SC_PARAMS,
    scratch_types=_COMMON_SCRATCH
    + [pltpu.VMEM_SHARED((N_PAD,), jnp.float32)]
    + _SEMS,
)(_make_edge_body(False))

_edge_callU = functools.partial(
    pl.kernel,
    out_type=(jax.ShapeDtypeStruct((N_PAD,), jnp.float32),
              jax.ShapeDtypeStruct((2 * N_PAD,), jnp.float32)),
    mesh=_SC_MESH,
    compiler_params=_SC_PARAMS,
    scratch_types=_COMMON_SCRATCH
    + [pltpu.VMEM((QSLICE,), jnp.float32)] * 7
    + [pltpu.VMEM_SHARED((N_PAD,), jnp.float32)]
    + _SEMS,
)(_make_edge_body(True))


def kernel(x, theta, node_bias, rate_log_scale, base_rate, conf_scale,
           delay_scale, capacity, edge_index, n_steps):
    theta2 = theta.reshape(GROUPS, LANES)
    conf2 = conf_scale.reshape(GROUPS, LANES)
    delay2 = delay_scale.reshape(GROUPS, LANES)
    src2 = edge_index[0].reshape(GROUPS, LANES)
    dst2 = edge_index[1].reshape(GROUPS, LANES)
    edata = _pack_edata(src2, dst2, theta2, conf2, delay2)
    edata = edata.reshape(3 * GROUPS, LANES)

    pad = N_PAD - N
    x_pad = jnp.pad(x, (0, pad))
    bias_pad = jnp.pad(node_bias, (0, pad))
    rls_pad = jnp.pad(rate_log_scale, (0, pad))
    base_pad = jnp.pad(base_rate, (0, pad))
    cap_pad = jnp.pad(capacity, (0, pad), constant_values=1.0)

    BISECT_TC_UPDATE = False
    if BISECT_TC_UPDATE:
        def _step_b(_, st):
            pr = _edge_call0(st, edata)
            return _node_update(
                st.reshape(ROWS_PAD, 128),
                pr[:N_PAD].reshape(ROWS_PAD, 128),
                pr[N_PAD:].reshape(ROWS_PAD, 128),
                bias_pad.reshape(ROWS_PAD, 128),
                rls_pad.reshape(ROWS_PAD, 128),
                base_pad.reshape(ROWS_PAD, 128),
                cap_pad.reshape(ROWS_PAD, 128)).reshape(N_PAD)

        st = lax.fori_loop(0, n_steps, _step_b, x_pad)
        return st[:N]

    parts0 = _edge_call0(x_pad, edata)

    def _step(_, carry):
        st, pr = carry
        st2, pr2 = _edge_callU(st, pr, edata, bias_pad, base_pad, rls_pad,
                               cap_pad)
        return (st2, pr2)

    st, pr = lax.fori_loop(1, n_steps, _step, (x_pad, parts0))

    final = _node_update(st.reshape(ROWS_PAD, 128),
                         pr[:N_PAD].reshape(ROWS_PAD, 128),
                         pr[N_PAD:].reshape(ROWS_PAD, 128),
                         bias_pad.reshape(ROWS_PAD, 128),
                         rls_pad.reshape(ROWS_PAD, 128),
                         base_pad.reshape(ROWS_PAD, 128),
                         cap_pad.reshape(ROWS_PAD, 128))
    return final.reshape(N_PAD)[:N]


# speedup vs baseline: 1.0028x; 1.0028x over previous
import functools

import jax
import jax.numpy as jnp
from jax import lax
from jax.experimental import pallas as pl
from jax.experimental.pallas import tpu as pltpu
from jax.experimental.pallas import tpu_sc as plsc

N = 100000
E = 6400000
DT = 0.1
EPS = 1e-5

LANES = 128
GROUPS = E // LANES
N_PAD = 100352
ROWS_PAD = N_PAD // 128
G_CHUNK = 8
N_CHUNKS = 196
NBUF = 4
SLICE = N_PAD // 16
NPIECE = 8
QSLICE = SLICE // NPIECE
HSLICE = QSLICE // 2


def _pack_body(src_ref, dst_ref, th_ref, cf_ref, dl_ref, out_ref):
    w = jnp.tanh(th_ref[...]) * cf_ref[...] * dl_ref[...]
    out_ref[:, 0, :] = src_ref[...]
    out_ref[:, 1, :] = dst_ref[...]
    out_ref[:, 2, :] = lax.bitcast_convert_type(w, jnp.int32)


def _pack_edata(src2, dst2, theta2, conf2, delay2):
    blk = 2000
    spec = pl.BlockSpec((blk, LANES), lambda i: (i, 0))
    return pl.pallas_call(
        _pack_body,
        grid=(GROUPS // blk,),
        in_specs=[spec] * 5,
        out_specs=pl.BlockSpec((blk, 3, LANES), lambda i: (i, 0, 0)),
        out_shape=jax.ShapeDtypeStruct((GROUPS, 3, LANES), jnp.int32),
    )(src2, dst2, theta2, conf2, delay2)


def _update_body(state_ref, p0_ref, p1_ref, bias_ref, rls_ref, base_ref,
                 cap_ref, out_ref):
    s = state_ref[...]
    infl = p0_ref[...] + p1_ref[...]
    drive = jnp.tanh(infl + bias_ref[...])
    rate = base_ref[...] * jnp.exp(rls_ref[...])
    cap = cap_ref[...]
    dx = rate * drive * s * (1.0 - s / jnp.clip(cap, EPS))
    out_ref[...] = jnp.clip(s + DT * dx, 0.0, cap)


def _node_update(state2, p0, p1, bias2, rls2, base2, cap2):
    return pl.pallas_call(
        _update_body,
        out_shape=jax.ShapeDtypeStruct((ROWS_PAD, 128), jnp.float32),
    )(state2, p0, p1, bias2, rls2, base2, cap2)


def _make_edge_body(with_update):
    def body(*refs):
        if with_update:
            (state_hbm, parts_hbm, edata_hbm, bias_hbm, base_hbm, rls_hbm,
             cap_hbm, stateout_hbm, partsout_hbm,
             state_v, ring_v, msg_v, stage_v,
             u_s, u_p0, u_p1, u_bias, u_base, u_rls, u_cap, infl_s,
             sem_state, sem_in0, sem_in1, sem_in2, sem_in3,
             sem_sc0, sem_sc1, sem_sc2, sem_sc3) = refs
        else:
            (state_hbm, edata_hbm, partsout_hbm,
             state_v, ring_v, msg_v, stage_v, infl_s,
             sem_state, sem_in0, sem_in1, sem_in2, sem_in3,
             sem_sc0, sem_sc1, sem_sc2, sem_sc3) = refs
        sem_in = [sem_in0, sem_in1, sem_in2, sem_in3]
        sem_sc = [sem_sc0, sem_sc1, sem_sc2, sem_sc3]
        cid = lax.axis_index("c")
        sid = lax.axis_index("s")
        wid = sid * 2 + cid
        n_g = jnp.where(wid < 10, 1568, 1560)
        base_g = 8 * (195 * wid + jnp.minimum(wid, 10))

        def _row0(c):
            return base_g + jnp.minimum(G_CHUNK * c, n_g - G_CHUNK)

        def _fire_in(c, b):
            pltpu.async_copy(edata_hbm.at[pl.ds(_row0(c) * 3, 3 * G_CHUNK)],
                             ring_v.at[b], sem_in[b])

        def _wait_in(c, b):
            pltpu.make_async_copy(edata_hbm.at[pl.ds(_row0(c) * 3, 3 * G_CHUNK)],
                                  ring_v.at[b], sem_in[b]).wait()

        def _fire_sc(b):
            for j in range(G_CHUNK):
                pltpu.async_copy(msg_v.at[b, j],
                                 infl_s.at[ring_v.at[b, 3 * j + 1]],
                                 sem_sc[b], add=True)

        def _drain_sc(b):
            for j in range(G_CHUNK):
                pltpu.make_async_copy(msg_v.at[b, j],
                                      infl_s.at[ring_v.at[b, 3 * j + 1]],
                                      sem_sc[b]).wait()

        if not with_update:
            state_cp = pltpu.async_copy(state_hbm, state_v, sem_state)

        zeros16 = jnp.zeros((16,), jnp.float32)

        def _zero_body(i, carry):
            stage_v[pl.ds(i * 16, 16)] = zeros16
            return carry

        def _zero_infl():
            lax.fori_loop(0, QSLICE // 16, _zero_body, 0)
            for q in range(NPIECE):
                pltpu.sync_copy(
                    stage_v,
                    infl_s.at[pl.ds(sid * SLICE + q * QSLICE, QSLICE)])

        if with_update:
            state_s = infl_s
            for q in range(NPIECE):
                off = sid * SLICE + q * QSLICE
                sl_q = pl.ds(off, QSLICE)
                cps = [
                    pltpu.async_copy(state_hbm.at[sl_q], u_s, sem_state),
                    pltpu.async_copy(parts_hbm.at[sl_q], u_p0, sem_state),
                    pltpu.async_copy(parts_hbm.at[pl.ds(N_PAD + off, QSLICE)],
                                     u_p1, sem_state),
                    pltpu.async_copy(bias_hbm.at[sl_q], u_bias, sem_state),
                    pltpu.async_copy(base_hbm.at[sl_q], u_base, sem_state),
                    pltpu.async_copy(rls_hbm.at[sl_q], u_rls, sem_state),
                    pltpu.async_copy(cap_hbm.at[sl_q], u_cap, sem_state),
                ]
                for cp in cps:
                    cp.wait()
                def _upd_body(i, carry):
                    sl = pl.ds(i * 16, 16)
                    s = u_s[sl]
                    y = u_p0[sl] + u_p1[sl] + u_bias[sl]
                    t = jnp.exp(-2.0 * jnp.abs(y))
                    mag = (1.0 - t) / (1.0 + t)
                    drive = jnp.where(y >= 0.0, mag, -mag)
                    rate = u_base[sl] * jnp.exp(u_rls[sl])
                    cap = u_cap[sl]
                    dx = rate * drive * s * (1.0 - s / jnp.maximum(cap, EPS))
                    stage_v[sl] = jnp.clip(s + DT * dx, 0.0, cap)
                    return carry

                lax.fori_loop(0, QSLICE // 16, _upd_body, 0)
                pltpu.sync_copy(stage_v, state_s.at[sl_q])
                pltpu.sync_copy(
                    stage_v.at[pl.ds(cid * HSLICE, HSLICE)],
                    stateout_hbm.at[pl.ds(off + cid * HSLICE, HSLICE)])
            plsc.subcore_barrier()
            pltpu.sync_copy(state_s, state_v)
            plsc.subcore_barrier()
            _zero_infl()
            plsc.subcore_barrier()
        else:
            _zero_infl()
            state_cp.wait()
            plsc.subcore_barrier()

        _fire_in(0, 0)
        _fire_in(1, 1)

        def _chunk_body(p, carry):
            for b in range(NBUF):
                c = NBUF * p + b
                bg = jnp.minimum(G_CHUNK * c, n_g - G_CHUNK)
                _wait_in(c, b)
                for j in range(G_CHUNK):
                    valid = (bg + j >= G_CHUNK * c).astype(jnp.float32)
                    for k in range(LANES // 16):
                        sl = pl.ds(k * 16, 16)
                        idx = ring_v[b, 3 * j, sl]
                        wv = plsc.bitcast(ring_v[b, 3 * j + 2, sl], jnp.float32)
                        vals = plsc.load_gather(state_v, [idx])
                        msg_v[b, j, sl] = vals * wv * valid
                b2 = (b + 2) % NBUF

                @pl.when(c >= 2)
                def _():
                    _drain_sc(b2)

                _fire_sc(b)

                @pl.when(c + 2 <= N_CHUNKS - 1)
                def _():
                    _fire_in(c + 2, b2)
            return carry

        lax.fori_loop(0, N_CHUNKS // NBUF, _chunk_body, 0)
        _drain_sc(2)
        _drain_sc(3)
        plsc.subcore_barrier()

        for q in range(NPIECE):
            off = sid * SLICE + q * QSLICE
            pltpu.sync_copy(infl_s.at[pl.ds(off, QSLICE)], stage_v)
            pltpu.sync_copy(stage_v,
                            partsout_hbm.at[pl.ds(cid * N_PAD + off, QSLICE)])

    return body


_SC_MESH = plsc.VectorSubcoreMesh(core_axis_name="c", subcore_axis_name="s")
_SC_PARAMS = pltpu.CompilerParams(needs_layout_passes=False)
_COMMON_SCRATCH = [
    pltpu.VMEM((N_PAD,), jnp.float32),
    pltpu.VMEM((NBUF, 3 * G_CHUNK, LANES), jnp.int32),
    pltpu.VMEM((NBUF, G_CHUNK, LANES), jnp.float32),
    pltpu.VMEM((QSLICE,), jnp.float32),
]
_SEMS = [pltpu.SemaphoreType.DMA] * 9

_edge_call0 = functools.partial(
    pl.kernel,
    out_type=jax.ShapeDtypeStruct((2 * N_PAD,), jnp.float32),
    mesh=_SC_MESH,
    compiler_params=_SC_PARAMS,
    scratch_types=_COMMON_SCRATCH
    + [pltpu.VMEM_SHARED((N_PAD,), jnp.float32)]
    + _SEMS,
)(_make_edge_body(False))

_edge_callU = functools.partial(
    pl.kernel,
    out_type=(jax.ShapeDtypeStruct((N_PAD,), jnp.float32),
              jax.ShapeDtypeStruct((2 * N_PAD,), jnp.float32)),
    mesh=_SC_MESH,
    compiler_params=_SC_PARAMS,
    scratch_types=_COMMON_SCRATCH
    + [pltpu.VMEM((QSLICE,), jnp.float32)] * 7
    + [pltpu.VMEM_SHARED((N_PAD,), jnp.float32)]
    + _SEMS,
)(_make_edge_body(True))


def kernel(x, theta, node_bias, rate_log_scale, base_rate, conf_scale,
           delay_scale, capacity, edge_index, n_steps):
    theta2 = theta.reshape(GROUPS, LANES)
    conf2 = conf_scale.reshape(GROUPS, LANES)
    delay2 = delay_scale.reshape(GROUPS, LANES)
    src2 = edge_index[0].reshape(GROUPS, LANES)
    dst2 = edge_index[1].reshape(GROUPS, LANES)
    edata = _pack_edata(src2, dst2, theta2, conf2, delay2)
    edata = edata.reshape(3 * GROUPS, LANES)

    pad = N_PAD - N
    x_pad = jnp.pad(x, (0, pad))
    bias_pad = jnp.pad(node_bias, (0, pad))
    rls_pad = jnp.pad(rate_log_scale, (0, pad))
    base_pad = jnp.pad(base_rate, (0, pad))
    cap_pad = jnp.pad(capacity, (0, pad), constant_values=1.0)

    BISECT_TC_UPDATE = True
    if BISECT_TC_UPDATE:
        def _step_b(_, st):
            pr = _edge_call0(st, edata)
            return _node_update(
                st.reshape(ROWS_PAD, 128),
                pr[:N_PAD].reshape(ROWS_PAD, 128),
                pr[N_PAD:].reshape(ROWS_PAD, 128),
                bias_pad.reshape(ROWS_PAD, 128),
                rls_pad.reshape(ROWS_PAD, 128),
                base_pad.reshape(ROWS_PAD, 128),
                cap_pad.reshape(ROWS_PAD, 128)).reshape(N_PAD)

        st = lax.fori_loop(0, n_steps, _step_b, x_pad)
        return st[:N]

    parts0 = _edge_call0(x_pad, edata)

    def _step(_, carry):
        st, pr = carry
        st2, pr2 = _edge_callU(st, pr, edata, bias_pad, base_pad, rls_pad,
                               cap_pad)
        return (st2, pr2)

    st, pr = lax.fori_loop(1, n_steps, _step, (x_pad, parts0))

    final = _node_update(st.reshape(ROWS_PAD, 128),
                         pr[:N_PAD].reshape(ROWS_PAD, 128),
                         pr[N_PAD:].reshape(ROWS_PAD, 128),
                         bias_pad.reshape(ROWS_PAD, 128),
                         rls_pad.reshape(ROWS_PAD, 128),
                         base_pad.reshape(ROWS_PAD, 128),
                         cap_pad.reshape(ROWS_PAD, 128))
    return final.reshape(N_PAD)[:N]


# speedup vs baseline: 1.2332x; 1.2297x over previous
import functools

import jax
import jax.numpy as jnp
from jax import lax
from jax.experimental import pallas as pl
from jax.experimental.pallas import tpu as pltpu
from jax.experimental.pallas import tpu_sc as plsc

N = 100000
E = 6400000
DT = 0.1
EPS = 1e-5

LANES = 128
GROUPS = E // LANES
N_PAD = 100352
ROWS_PAD = N_PAD // 128
G_CHUNK = 8
N_CHUNKS = 196
NBUF = 4
SLICE = N_PAD // 16
NPIECE = 8
QSLICE = SLICE // NPIECE
HSLICE = QSLICE // 2


def _w_body(th_ref, cf_ref, dl_ref, out_ref):
    out_ref[...] = jnp.tanh(th_ref[...]) * cf_ref[...] * dl_ref[...]


def _compute_w(theta2, conf2, delay2):
    blk = 2000
    spec = pl.BlockSpec((blk, LANES), lambda i: (i, 0))
    return pl.pallas_call(
        _w_body,
        grid=(GROUPS // blk,),
        in_specs=[spec] * 3,
        out_specs=spec,
        out_shape=jax.ShapeDtypeStruct((GROUPS, LANES), jnp.float32),
    )(theta2, conf2, delay2)


def _update_body(state_ref, p0_ref, p1_ref, bias_ref, rls_ref, base_ref,
                 cap_ref, out_ref):
    s = state_ref[...]
    infl = p0_ref[...] + p1_ref[...]
    drive = jnp.tanh(infl + bias_ref[...])
    rate = base_ref[...] * jnp.exp(rls_ref[...])
    cap = cap_ref[...]
    dx = rate * drive * s * (1.0 - s / jnp.clip(cap, EPS))
    out_ref[...] = jnp.clip(s + DT * dx, 0.0, cap)


def _node_update(state2, p0, p1, bias2, rls2, base2, cap2):
    return pl.pallas_call(
        _update_body,
        out_shape=jax.ShapeDtypeStruct((ROWS_PAD, 128), jnp.float32),
    )(state2, p0, p1, bias2, rls2, base2, cap2)


def _make_edge_body(with_update):
    def body(*refs):
        if with_update:
            (state_hbm, parts_hbm, src_hbm, dst_hbm, w_hbm, bias_hbm,
             base_hbm, rls_hbm, cap_hbm, stateout_hbm, partsout_hbm,
             state_v, src_v, dst_v, w_v, msg_v, stage_v,
             u_s, u_p0, u_p1, u_bias, u_base, u_rls, u_cap, infl_s,
             sem_state, sem_in0, sem_in1, sem_in2, sem_in3,
             sem_sc0, sem_sc1, sem_sc2, sem_sc3) = refs
        else:
            (state_hbm, src_hbm, dst_hbm, w_hbm, partsout_hbm,
             state_v, src_v, dst_v, w_v, msg_v, stage_v, infl_s,
             sem_state, sem_in0, sem_in1, sem_in2, sem_in3,
             sem_sc0, sem_sc1, sem_sc2, sem_sc3) = refs
        sem_in = [sem_in0, sem_in1, sem_in2, sem_in3]
        sem_sc = [sem_sc0, sem_sc1, sem_sc2, sem_sc3]
        cid = lax.axis_index("c")
        sid = lax.axis_index("s")
        wid = sid * 2 + cid
        n_g = jnp.where(wid < 10, 1568, 1560)
        base_g = 8 * (195 * wid + jnp.minimum(wid, 10))

        def _row0(c):
            return base_g + jnp.minimum(G_CHUNK * c, n_g - G_CHUNK)

        def _fire_in(c, b):
            row0 = _row0(c)
            pltpu.async_copy(src_hbm.at[pl.ds(row0, G_CHUNK)], src_v.at[b],
                             sem_in[b])
            pltpu.async_copy(dst_hbm.at[pl.ds(row0, G_CHUNK)], dst_v.at[b],
                             sem_in[b])
            pltpu.async_copy(w_hbm.at[pl.ds(row0, G_CHUNK)], w_v.at[b],
                             sem_in[b])

        def _wait_in(c, b):
            row0 = _row0(c)
            pltpu.make_async_copy(src_hbm.at[pl.ds(row0, G_CHUNK)],
                                  src_v.at[b], sem_in[b]).wait()
            pltpu.make_async_copy(dst_hbm.at[pl.ds(row0, G_CHUNK)],
                                  dst_v.at[b], sem_in[b]).wait()
            pltpu.make_async_copy(w_hbm.at[pl.ds(row0, G_CHUNK)],
                                  w_v.at[b], sem_in[b]).wait()

        def _fire_sc(b):
            for j in range(G_CHUNK):
                pltpu.async_copy(msg_v.at[b, j],
                                 infl_s.at[dst_v.at[b, j]],
                                 sem_sc[b], add=True)

        def _drain_sc(b):
            for j in range(G_CHUNK):
                pltpu.make_async_copy(msg_v.at[b, j],
                                      infl_s.at[dst_v.at[b, j]],
                                      sem_sc[b]).wait()

        if not with_update:
            state_cp = pltpu.async_copy(state_hbm, state_v, sem_state)

        zeros16 = jnp.zeros((16,), jnp.float32)

        def _zero_body(i, carry):
            stage_v[pl.ds(i * 16, 16)] = zeros16
            return carry

        def _zero_infl():
            lax.fori_loop(0, QSLICE // 16, _zero_body, 0)
            for q in range(NPIECE):
                pltpu.sync_copy(
                    stage_v,
                    infl_s.at[pl.ds(sid * SLICE + q * QSLICE, QSLICE)])

        if with_update:
            state_s = infl_s
            for q in range(NPIECE):
                off = sid * SLICE + q * QSLICE
                sl_q = pl.ds(off, QSLICE)
                cps = [
                    pltpu.async_copy(state_hbm.at[sl_q], u_s, sem_state),
                    pltpu.async_copy(parts_hbm.at[sl_q], u_p0, sem_state),
                    pltpu.async_copy(parts_hbm.at[pl.ds(N_PAD + off, QSLICE)],
                                     u_p1, sem_state),
                    pltpu.async_copy(bias_hbm.at[sl_q], u_bias, sem_state),
                    pltpu.async_copy(base_hbm.at[sl_q], u_base, sem_state),
                    pltpu.async_copy(rls_hbm.at[sl_q], u_rls, sem_state),
                    pltpu.async_copy(cap_hbm.at[sl_q], u_cap, sem_state),
                ]
                for cp in cps:
                    cp.wait()
                def _upd_body(i, carry):
                    sl = pl.ds(i * 16, 16)
                    s = u_s[sl]
                    y = u_p0[sl] + u_p1[sl] + u_bias[sl]
                    t = jnp.exp(-2.0 * jnp.abs(y))
                    mag = (1.0 - t) / (1.0 + t)
                    drive = jnp.where(y >= 0.0, mag, -mag)
                    rate = u_base[sl] * jnp.exp(u_rls[sl])
                    cap = u_cap[sl]
                    dx = rate * drive * s * (1.0 - s / jnp.maximum(cap, EPS))
                    stage_v[sl] = jnp.clip(s + DT * dx, 0.0, cap)
                    return carry

                lax.fori_loop(0, QSLICE // 16, _upd_body, 0)
                pltpu.sync_copy(stage_v, state_s.at[sl_q])
                pltpu.sync_copy(
                    stage_v.at[pl.ds(cid * HSLICE, HSLICE)],
                    stateout_hbm.at[pl.ds(off + cid * HSLICE, HSLICE)])
            plsc.subcore_barrier()
            pltpu.sync_copy(state_s, state_v)
            plsc.subcore_barrier()
            _zero_infl()
            plsc.subcore_barrier()
        else:
            _zero_infl()
            state_cp.wait()
            plsc.subcore_barrier()

        _fire_in(0, 0)
        _fire_in(1, 1)

        def _chunk_body(p, carry):
            for b in range(NBUF):
                c = NBUF * p + b
                bg = jnp.minimum(G_CHUNK * c, n_g - G_CHUNK)
                _wait_in(c, b)
                for j in range(G_CHUNK):
                    valid = (bg + j >= G_CHUNK * c).astype(jnp.float32)
                    for k in range(LANES // 16):
                        sl = pl.ds(k * 16, 16)
                        idx = src_v[b, j, sl]
                        wv = w_v[b, j, sl]
                        vals = plsc.load_gather(state_v, [idx])
                        msg_v[b, j, sl] = vals * wv * valid
                b2 = (b + 2) % NBUF

                @pl.when(c >= 2)
                def _():
                    _drain_sc(b2)

                _fire_sc(b)

                @pl.when(c + 2 <= N_CHUNKS - 1)
                def _():
                    _fire_in(c + 2, b2)
            return carry

        lax.fori_loop(0, N_CHUNKS // NBUF, _chunk_body, 0)
        _drain_sc(2)
        _drain_sc(3)
        plsc.subcore_barrier()

        for q in range(NPIECE):
            off = sid * SLICE + q * QSLICE
            pltpu.sync_copy(infl_s.at[pl.ds(off, QSLICE)], stage_v)
            pltpu.sync_copy(stage_v,
                            partsout_hbm.at[pl.ds(cid * N_PAD + off, QSLICE)])

    return body


_SC_MESH = plsc.VectorSubcoreMesh(core_axis_name="c", subcore_axis_name="s")
_SC_PARAMS = pltpu.CompilerParams(needs_layout_passes=False)
_COMMON_SCRATCH = [
    pltpu.VMEM((N_PAD,), jnp.float32),
    pltpu.VMEM((NBUF, G_CHUNK, LANES), jnp.int32),
    pltpu.VMEM((NBUF, G_CHUNK, LANES), jnp.int32),
    pltpu.VMEM((NBUF, G_CHUNK, LANES), jnp.float32),
    pltpu.VMEM((NBUF, G_CHUNK, LANES), jnp.float32),
    pltpu.VMEM((QSLICE,), jnp.float32),
]
_SEMS = [pltpu.SemaphoreType.DMA] * 9

_edge_call0 = functools.partial(
    pl.kernel,
    out_type=jax.ShapeDtypeStruct((2 * N_PAD,), jnp.float32),
    mesh=_SC_MESH,
    compiler_params=_SC_PARAMS,
    scratch_types=_COMMON_SCRATCH
    + [pltpu.VMEM_SHARED((N_PAD,), jnp.float32)]
    + _SEMS,
)(_make_edge_body(False))

_edge_callU = functools.partial(
    pl.kernel,
    out_type=(jax.ShapeDtypeStruct((N_PAD,), jnp.float32),
              jax.ShapeDtypeStruct((2 * N_PAD,), jnp.float32)),
    mesh=_SC_MESH,
    compiler_params=_SC_PARAMS,
    scratch_types=_COMMON_SCRATCH
    + [pltpu.VMEM((QSLICE,), jnp.float32)] * 7
    + [pltpu.VMEM_SHARED((N_PAD,), jnp.float32)]
    + _SEMS,
)(_make_edge_body(True))


def kernel(x, theta, node_bias, rate_log_scale, base_rate, conf_scale,
           delay_scale, capacity, edge_index, n_steps):
    theta2 = theta.reshape(GROUPS, LANES)
    conf2 = conf_scale.reshape(GROUPS, LANES)
    delay2 = delay_scale.reshape(GROUPS, LANES)
    src2 = edge_index[0].reshape(GROUPS, LANES)
    dst2 = edge_index[1].reshape(GROUPS, LANES)
    w2 = _compute_w(theta2, conf2, delay2)

    pad = N_PAD - N
    x_pad = jnp.pad(x, (0, pad))
    bias_pad = jnp.pad(node_bias, (0, pad))
    rls_pad = jnp.pad(rate_log_scale, (0, pad))
    base_pad = jnp.pad(base_rate, (0, pad))
    cap_pad = jnp.pad(capacity, (0, pad), constant_values=1.0)

    BISECT_TC_UPDATE = False
    if BISECT_TC_UPDATE:
        def _step_b(_, st):
            pr = _edge_call0(st, src2, dst2, w2)
            return _node_update(
                st.reshape(ROWS_PAD, 128),
                pr[:N_PAD].reshape(ROWS_PAD, 128),
                pr[N_PAD:].reshape(ROWS_PAD, 128),
                bias_pad.reshape(ROWS_PAD, 128),
                rls_pad.reshape(ROWS_PAD, 128),
                base_pad.reshape(ROWS_PAD, 128),
                cap_pad.reshape(ROWS_PAD, 128)).reshape(N_PAD)

        st = lax.fori_loop(0, n_steps, _step_b, x_pad)
        return st[:N]

    parts0 = _edge_call0(x_pad, src2, dst2, w2)

    def _step(_, carry):
        st, pr = carry
        st2, pr2 = _edge_callU(st, pr, src2, dst2, w2, bias_pad, base_pad,
                               rls_pad, cap_pad)
        return (st2, pr2)

    st, pr = lax.fori_loop(1, n_steps, _step, (x_pad, parts0))

    final = _node_update(st.reshape(ROWS_PAD, 128),
                         pr[:N_PAD].reshape(ROWS_PAD, 128),
                         pr[N_PAD:].reshape(ROWS_PAD, 128),
                         bias_pad.reshape(ROWS_PAD, 128),
                         rls_pad.reshape(ROWS_PAD, 128),
                         base_pad.reshape(ROWS_PAD, 128),
                         cap_pad.reshape(ROWS_PAD, 128))
    return final.reshape(N_PAD)[:N]
